# Initial kernel scaffold; baseline (speedup 1.0000x reference)
import jax
import jax.numpy as jnp
from jax.experimental import pallas as pl


def kernel(x, edge_index, batch, W1, b1, W2, b2):
    raise NotImplementedError("write your pallas kernel here")


# speedup vs baseline: 10.9230x; 10.9230x over previous
import functools

import jax
import jax.numpy as jnp
from jax import lax
from jax.experimental import pallas as pl
from jax.experimental.pallas import tpu as pltpu
from jax.experimental.pallas import tpu_sc as plsc

NC = 2
NS = 16
NW = NC * NS
LANES = 16
K = 128
ZK = 128


def _mesh():
    return plsc.VectorSubcoreMesh(
        core_axis_name="c", subcore_axis_name="s", num_cores=NC, num_subcores=NS
    )


def _deg_body(n_acc, ep, dstp, out, dst_v, ones_v, zvec, deg_acc):
    c = lax.axis_index("c")
    s = lax.axis_index("s")
    wid = c * NS + s
    rp = n_acc // NS

    def fill(i, _):
        zvec[pl.ds(i * LANES, LANES)] = jnp.zeros((LANES,), jnp.float32)
        ones_v[pl.ds((i % (K // LANES)) * LANES, LANES)] = jnp.ones(
            (LANES,), jnp.float32
        )
        return 0

    lax.fori_loop(0, rp // LANES, fill, 0)
    pltpu.sync_copy(zvec, deg_acc.at[pl.ds(s * rp, rp)])
    plsc.subcore_barrier()

    def chunk(k, _):
        off = pl.multiple_of(wid * ep + k * K, 8)
        pltpu.sync_copy(dstp.at[pl.ds(off, K)], dst_v)
        pltpu.sync_copy(ones_v, deg_acc.at[dst_v], add=True)
        return 0

    lax.fori_loop(0, ep // K, chunk, 0)
    plsc.subcore_barrier()
    pltpu.sync_copy(deg_acc.at[pl.ds(s * rp, rp)], out.at[c, pl.ds(s * rp, rp)])


def _agg_body(n_acc, ep, d, g, srcp, dstp, out, src_v, dst_v, rows_v, zbuf, sem, acc):
    c = lax.axis_index("c")
    s = lax.axis_index("s")
    wid = c * NS + s
    rp = n_acc // NS

    def fill(i, _):
        zbuf[i // (d // LANES), pl.ds((i % (d // LANES)) * LANES, LANES)] = (
            jnp.zeros((LANES,), jnp.float32)
        )
        return 0

    lax.fori_loop(0, ZK * d // LANES, fill, 0)
    for z in range(rp // ZK):
        pltpu.sync_copy(zbuf, acc.at[pl.ds(s * rp + z * ZK, ZK)])
    plsc.subcore_barrier()

    def chunk(k, _):
        off = pl.multiple_of(wid * ep + k * K, 8)
        pltpu.sync_copy(srcp.at[pl.ds(off, K)], src_v)
        pltpu.sync_copy(dstp.at[pl.ds(off, K)], dst_v)
        pltpu.async_copy(g.at[src_v], rows_v, sem).wait()
        pltpu.sync_copy(rows_v, acc.at[dst_v], add=True)
        return 0

    lax.fori_loop(0, ep // K, chunk, 0)
    plsc.subcore_barrier()
    pltpu.sync_copy(acc.at[pl.ds(s * rp, rp)], out.at[c, pl.ds(s * rp, rp)])


def _tca_body(x_ref, w_ref, degp_ref, g_ref):
    deg = degp_ref[0, :] + degp_ref[1, :] + 1.0
    dinv = lax.rsqrt(deg)[:, None]
    h = jnp.dot(x_ref[...], w_ref[...], preferred_element_type=jnp.float32)
    g_ref[...] = h * dinv


def _tcb_body(p_ref, g_ref, degp_ref, w_ref, b_ref, out_ref):
    deg = degp_ref[0, :] + degp_ref[1, :] + 1.0
    dinv = lax.rsqrt(deg)[:, None]
    t = dinv * (p_ref[0] + p_ref[1] + g_ref[...]) + b_ref[...]
    o1 = jnp.where(t >= 0, t, 0.01 * t)
    h2 = jnp.dot(o1, w_ref[...], preferred_element_type=jnp.float32)
    out_ref[...] = h2 * dinv


def _tcc_body(p_ref, g_ref, degp_ref, b_ref, out_ref):
    deg = degp_ref[0, :] + degp_ref[1, :] + 1.0
    dinv = lax.rsqrt(deg)[:, None]
    t = dinv * (p_ref[0] + p_ref[1] + g_ref[...]) + b_ref[...]
    out_ref[...] = jnp.where(t >= 0, t, 0.01 * t)


def kernel(x, edge_index, batch, W1, b1, W2, b2):
    n, d = x.shape
    e = edge_index.shape[1]

    ep = -(-e // (NW * K)) * K
    e_pad = ep * NW
    n_acc = -(-n // (NS * ZK)) * NS * ZK
    src = edge_index[0].astype(jnp.int32)
    dst = edge_index[1].astype(jnp.int32)
    pad = e_pad - e
    srcp = jnp.concatenate([src, jnp.zeros((pad,), jnp.int32)])
    dstp = jnp.concatenate([dst, jnp.full((pad,), n, jnp.int32)])
    xp = jnp.concatenate([x, jnp.zeros((n_acc - n, d), x.dtype)])

    deg_kernel = pl.kernel(
        functools.partial(_deg_body, n_acc, ep),
        out_type=jax.ShapeDtypeStruct((NC, n_acc), jnp.float32),
        mesh=_mesh(),
        scratch_types={
            "dst_v": pltpu.VMEM((K,), jnp.int32),
            "ones_v": pltpu.VMEM((K,), jnp.float32),
            "zvec": pltpu.VMEM((n_acc // NS,), jnp.float32),
            "deg_acc": pltpu.MemorySpace.VMEM_SHARED((n_acc,), jnp.float32),
        },
        name="gcn_sc_degree",
    )

    agg_kernel = pl.kernel(
        functools.partial(_agg_body, n_acc, ep, d),
        out_type=jax.ShapeDtypeStruct((NC, n_acc, d), jnp.float32),
        mesh=_mesh(),
        scratch_types={
            "src_v": pltpu.VMEM((K,), jnp.int32),
            "dst_v": pltpu.VMEM((K,), jnp.int32),
            "rows_v": pltpu.VMEM((K, d), jnp.float32),
            "zbuf": pltpu.VMEM((ZK, d), jnp.float32),
            "sem": pltpu.SemaphoreType.DMA,
            "acc": pltpu.MemorySpace.VMEM_SHARED((n_acc, d), jnp.float32),
        },
        name="gcn_sc_scatter",
    )

    br = 2048
    grid = (n_acc // br,)
    tca = pl.pallas_call(
        _tca_body,
        grid=grid,
        in_specs=[
            pl.BlockSpec((br, d), lambda i: (i, 0)),
            pl.BlockSpec((d, d), lambda i: (0, 0)),
            pl.BlockSpec((NC, br), lambda i: (0, i)),
        ],
        out_specs=pl.BlockSpec((br, d), lambda i: (i, 0)),
        out_shape=jax.ShapeDtypeStruct((n_acc, d), jnp.float32),
        name="gcn_tc_g1",
    )
    tcb = pl.pallas_call(
        _tcb_body,
        grid=grid,
        in_specs=[
            pl.BlockSpec((NC, br, d), lambda i: (0, i, 0)),
            pl.BlockSpec((br, d), lambda i: (i, 0)),
            pl.BlockSpec((NC, br), lambda i: (0, i)),
            pl.BlockSpec((d, d), lambda i: (0, 0)),
            pl.BlockSpec((1, d), lambda i: (0, 0)),
        ],
        out_specs=pl.BlockSpec((br, d), lambda i: (i, 0)),
        out_shape=jax.ShapeDtypeStruct((n_acc, d), jnp.float32),
        name="gcn_tc_layer1",
    )
    tcc = pl.pallas_call(
        _tcc_body,
        grid=grid,
        in_specs=[
            pl.BlockSpec((NC, br, d), lambda i: (0, i, 0)),
            pl.BlockSpec((br, d), lambda i: (i, 0)),
            pl.BlockSpec((NC, br), lambda i: (0, i)),
            pl.BlockSpec((1, d), lambda i: (0, 0)),
        ],
        out_specs=pl.BlockSpec((br, d), lambda i: (i, 0)),
        out_shape=jax.ShapeDtypeStruct((n_acc, d), jnp.float32),
        name="gcn_tc_layer2",
    )

    degp = deg_kernel(dstp)
    g1 = tca(xp, W1, degp)
    p1 = agg_kernel(g1, srcp, dstp)
    g2 = tcb(p1, g1, degp, W2, b1.reshape(1, d))
    p2 = agg_kernel(g2, srcp, dstp)
    out = tcc(p2, g2, degp, b2.reshape(1, d))
    return out[:n]
